# baseline (speedup 1.0000x reference)
import functools

import jax
import jax.numpy as jnp
from jax import lax
from jax.experimental import pallas as pl
from jax.experimental.pallas import tpu as pltpu
from jax.experimental.pallas import tpu_sc as plsc

B = 16384
HALF = B // 2
VOCAB = 1000
TAB = 1024
EMB = 128
HID = 8

TILE_B = 2048
NC = 2
NS = 16
NW = NC * NS
BPW = HALF // NW
CH = 128
NCH = BPW // CH


def _table_body(emb_ref, w_ref, t_ref):
    t_ref[...] = jnp.dot(emb_ref[...], w_ref[...],
                         preferred_element_type=jnp.float32,
                         precision=lax.Precision.HIGHEST)


def _sc_gather_body(t_hbm, x0_hbm, x1_hbm, g0_hbm, g1_hbm,
                    idx0_v, idx1_v, rows0_v, rows1_v, sem):
    c = lax.axis_index("c")
    s = lax.axis_index("s")
    wid = s * NC + c
    pltpu.sync_copy(x0_hbm.at[pl.ds(wid * NCH, NCH)], idx0_v)
    pltpu.sync_copy(x1_hbm.at[pl.ds(wid * NCH, NCH)], idx1_v)
    copies = []
    for j in range(NCH):
        copies.append(pltpu.async_copy(
            t_hbm.at[idx0_v.at[j]], rows0_v.at[pl.ds(j * CH, CH)], sem))
        copies.append(pltpu.async_copy(
            t_hbm.at[idx1_v.at[j]], rows1_v.at[pl.ds(j * CH, CH)], sem))
    for cp in copies:
        cp.wait()
    base = wid * BPW
    pltpu.sync_copy(rows0_v, g0_hbm.at[pl.ds(base, BPW)])
    pltpu.sync_copy(rows1_v, g1_hbm.at[pl.ds(base, BPW)])


_sc_gather = functools.partial(
    pl.kernel,
    out_type=(
        jax.ShapeDtypeStruct((HALF, 16), jnp.float32),
        jax.ShapeDtypeStruct((HALF, 16), jnp.float32),
    ),
    mesh=plsc.VectorSubcoreMesh(core_axis_name="c", subcore_axis_name="s"),
    compiler_params=pltpu.CompilerParams(use_tc_tiling_on_sc=False),
    scratch_types=[
        pltpu.VMEM((NCH, CH), jnp.int32),
        pltpu.VMEM((NCH, CH), jnp.int32),
        pltpu.VMEM((BPW, 16), jnp.float32),
        pltpu.VMEM((BPW, 16), jnp.float32),
        pltpu.SemaphoreType.DMA,
    ],
)(_sc_gather_body)


def _mlp_body(g0_ref, g1_ref, b1_ref, w_ref, b2_ref, out_ref):
    g0 = g0_ref[...]
    g1 = g1_ref[...]
    hpre = g0[:, :HID] + g1[:, HID:2 * HID] + b1_ref[...]
    h = jnp.tanh(hpre).astype(jnp.bfloat16)
    out_ref[...] = (
        jnp.dot(h, w_ref[...], preferred_element_type=jnp.float32)
        + b2_ref[...])


def _mlp_body2(prev_ref, g0_ref, g1_ref, b1_ref, w_ref, b2_ref, out_ref):
    del prev_ref
    _mlp_body(g0_ref, g1_ref, b1_ref, w_ref, b2_ref, out_ref)


def kernel(x, emb, fc1_w, fc1_b, fc2_w, fc2_b):
    x = x.astype(jnp.int32)
    w_pack = jnp.concatenate(
        [fc1_w[:, :EMB].T, fc1_w[:, EMB:].T], axis=1)
    emb_pad = jnp.pad(emb, ((0, TAB - VOCAB), (0, 0)))
    table = pl.pallas_call(
        _table_body,
        out_shape=jax.ShapeDtypeStruct((TAB, 16), jnp.float32),
    )(emb_pad, w_pack)

    xr = x.reshape(2, HALF // CH, CH, 2)
    g0a, g1a = _sc_gather(table, xr[0, :, :, 0], xr[0, :, :, 1])
    g0b, g1b = _sc_gather(table, xr[1, :, :, 0], xr[1, :, :, 1])

    w2t = fc2_w.T.astype(jnp.bfloat16)
    b1 = fc1_b.reshape(1, HID)
    b2 = fc2_b.reshape(1, VOCAB)
    n_half = HALF // TILE_B
    common_specs = [
        pl.BlockSpec((TILE_B, 16), lambda i: (i, 0)),
        pl.BlockSpec((TILE_B, 16), lambda i: (i, 0)),
        pl.BlockSpec((1, HID), lambda i: (0, 0)),
        pl.BlockSpec((HID, VOCAB), lambda i: (0, 0)),
        pl.BlockSpec((1, VOCAB), lambda i: (0, 0)),
    ]
    out0 = pl.pallas_call(
        _mlp_body,
        grid=(n_half,),
        in_specs=common_specs,
        out_specs=pl.BlockSpec((TILE_B, VOCAB), lambda i: (i, 0)),
        out_shape=jax.ShapeDtypeStruct((B, VOCAB), jnp.float32),
    )(g0a, g1a, b1, w2t, b2)
    out = pl.pallas_call(
        _mlp_body2,
        grid=(n_half,),
        in_specs=[pl.BlockSpec(memory_space=pl.ANY)] + common_specs,
        out_specs=pl.BlockSpec((TILE_B, VOCAB),
                               lambda i, nh=n_half: (i + nh, 0)),
        out_shape=jax.ShapeDtypeStruct((B, VOCAB), jnp.float32),
        input_output_aliases={0: 0},
    )(out0, g0b, g1b, b1, w2t, b2)
    return out


# speedup vs baseline: 1.0184x; 1.0184x over previous
import functools

import jax
import jax.numpy as jnp
from jax import lax
from jax.experimental import pallas as pl
from jax.experimental.pallas import tpu as pltpu
from jax.experimental.pallas import tpu_sc as plsc

B = 16384
VOCAB = 1000
TAB = 1024
EMB = 128
HID = 8
NIDX = 2 * B

TILE_B = 2048
NC = 2
NS = 16
NW = NC * NS
BPW = NIDX // NW
CH = 128
NCH = BPW // CH


def _table_body(emb_ref, w_ref, t_ref):
    t_ref[...] = jnp.dot(emb_ref[...], w_ref[...],
                         preferred_element_type=jnp.float32,
                         precision=lax.Precision.HIGHEST)


def _sc_gather_body(t_hbm, xf_hbm, g_hbm, idx_v, rows_v, sem):
    c = lax.axis_index("c")
    s = lax.axis_index("s")
    wid = s * NC + c
    pltpu.sync_copy(xf_hbm.at[pl.ds(wid * NCH, NCH)], idx_v)
    copies = [
        pltpu.async_copy(
            t_hbm.at[idx_v.at[j]], rows_v.at[pl.ds(j * CH, CH)], sem)
        for j in range(NCH)
    ]
    for cp in copies:
        cp.wait()
    pltpu.sync_copy(rows_v, g_hbm.at[pl.ds(wid * BPW, BPW)])


_sc_gather = functools.partial(
    pl.kernel,
    out_type=jax.ShapeDtypeStruct((NIDX, 16), jnp.float32),
    mesh=plsc.VectorSubcoreMesh(core_axis_name="c", subcore_axis_name="s"),
    compiler_params=pltpu.CompilerParams(use_tc_tiling_on_sc=False),
    scratch_types=[
        pltpu.VMEM((NCH, CH), jnp.int32),
        pltpu.VMEM((BPW, 16), jnp.float32),
        pltpu.SemaphoreType.DMA,
    ],
)(_sc_gather_body)


def _mlp_body(g_ref, b1_ref, w_ref, b2_ref, out_ref):
    g = g_ref[...]
    hpre = g[:, :HID] + g[:, 24:24 + HID] + b1_ref[...]
    h = jnp.tanh(hpre).astype(jnp.bfloat16)
    out_ref[...] = (
        jnp.dot(h, w_ref[...], preferred_element_type=jnp.float32)
        + b2_ref[...])


def kernel(x, emb, fc1_w, fc1_b, fc2_w, fc2_b):
    x = x.astype(jnp.int32)
    w_pack = jnp.concatenate(
        [fc1_w[:, :EMB].T, fc1_w[:, EMB:].T], axis=1)
    emb_pad = jnp.pad(emb, ((0, TAB - VOCAB), (0, 0)))
    table = pl.pallas_call(
        _table_body,
        out_shape=jax.ShapeDtypeStruct((TAB, 16), jnp.float32),
    )(emb_pad, w_pack)

    xf = x.reshape(NIDX // CH, CH)
    g = _sc_gather(table, xf).reshape(B, 32)

    w2t = fc2_w.T.astype(jnp.bfloat16)
    b1 = fc1_b.reshape(1, HID)
    b2 = fc2_b.reshape(1, VOCAB)
    out = pl.pallas_call(
        _mlp_body,
        grid=(B // TILE_B,),
        in_specs=[
            pl.BlockSpec((TILE_B, 32), lambda i: (i, 0)),
            pl.BlockSpec((1, HID), lambda i: (0, 0)),
            pl.BlockSpec((HID, VOCAB), lambda i: (0, 0)),
            pl.BlockSpec((1, VOCAB), lambda i: (0, 0)),
        ],
        out_specs=pl.BlockSpec((TILE_B, VOCAB), lambda i: (i, 0)),
        out_shape=jax.ShapeDtypeStruct((B, VOCAB), jnp.float32),
    )(g, b1, w2t, b2)
    return out


# speedup vs baseline: 1.8726x; 1.8389x over previous
import functools

import jax
import jax.numpy as jnp
from jax import lax
from jax.experimental import pallas as pl
from jax.experimental.pallas import tpu as pltpu
from jax.experimental.pallas import tpu_sc as plsc

B = 16384
VOCAB = 1000
TAB = 1024
EMB = 128
HID = 8
NIDX = 2 * B

TILE_B = 2048
NC = 2
NS = 16
NW = NC * NS
BPW = NIDX // NW
CH = 128
NCH = BPW // CH


def _table_body(emb_ref, w_ref, t_ref):
    e = emb_ref[...]
    w = w_ref[...]
    pa = lax.dot_general(e, w[:, :EMB], (((1,), (1,)), ((), ())),
                         preferred_element_type=jnp.float32,
                         precision=lax.Precision.HIGHEST)
    pb = lax.dot_general(e, w[:, EMB:], (((1,), (1,)), ((), ())),
                         preferred_element_type=jnp.float32,
                         precision=lax.Precision.HIGHEST)
    t_ref[...] = jnp.zeros((TAB, 16), jnp.float32)
    t_ref[:VOCAB, :] = jnp.concatenate([pa, pb], axis=1)


def _sc_gather_body(t_hbm, xf_hbm, g_hbm, idx_v, rows_v, sem):
    c = lax.axis_index("c")
    s = lax.axis_index("s")
    wid = s * NC + c
    pltpu.sync_copy(xf_hbm.at[pl.ds(wid * NCH, NCH)], idx_v)
    copies = [
        pltpu.async_copy(
            t_hbm.at[idx_v.at[j]], rows_v.at[pl.ds(j * CH, CH)], sem)
        for j in range(NCH)
    ]
    for cp in copies:
        cp.wait()
    pltpu.sync_copy(rows_v, g_hbm.at[pl.ds(wid * BPW, BPW)])


_sc_gather = functools.partial(
    pl.kernel,
    out_type=jax.ShapeDtypeStruct((NIDX, 16), jnp.float32),
    mesh=plsc.VectorSubcoreMesh(core_axis_name="c", subcore_axis_name="s"),
    compiler_params=pltpu.CompilerParams(use_tc_tiling_on_sc=False),
    scratch_types=[
        pltpu.VMEM((NCH, CH), jnp.int32),
        pltpu.VMEM((BPW, 16), jnp.float32),
        pltpu.SemaphoreType.DMA,
    ],
)(_sc_gather_body)


def _mlp_body(g_ref, b1_ref, w_ref, b2_ref, out_ref):
    g = g_ref[...]
    hpre = g[:, :HID] + g[:, 24:24 + HID] + b1_ref[...]
    h = jnp.tanh(hpre).astype(jnp.bfloat16)
    out_ref[...] = (
        lax.dot_general(w_ref[...], h, (((1,), (1,)), ((), ())),
                        preferred_element_type=jnp.float32)
        + b2_ref[...])


def kernel(x, emb, fc1_w, fc1_b, fc2_w, fc2_b):
    table = pl.pallas_call(
        _table_body,
        out_shape=jax.ShapeDtypeStruct((TAB, 16), jnp.float32),
    )(emb, fc1_w)

    xf = x.astype(jnp.int32).reshape(NIDX // CH, CH)
    g = _sc_gather(table, xf).reshape(B, 32)

    w2 = fc2_w.astype(jnp.bfloat16)
    b1 = fc1_b.reshape(1, HID)
    b2 = fc2_b.reshape(VOCAB, 1)
    out_t = pl.pallas_call(
        _mlp_body,
        grid=(B // TILE_B,),
        in_specs=[
            pl.BlockSpec((TILE_B, 32), lambda i: (i, 0)),
            pl.BlockSpec((1, HID), lambda i: (0, 0)),
            pl.BlockSpec((VOCAB, HID), lambda i: (0, 0)),
            pl.BlockSpec((VOCAB, 1), lambda i: (0, 0)),
        ],
        out_specs=pl.BlockSpec((VOCAB, TILE_B), lambda i: (0, i)),
        out_shape=jax.ShapeDtypeStruct((VOCAB, B), jnp.float32),
    )(g, b1, w2, b2)
    return out_t.T


# speedup vs baseline: 2.4553x; 1.3111x over previous
import functools

import jax
import jax.numpy as jnp
from jax import lax
from jax.experimental import pallas as pl
from jax.experimental.pallas import tpu as pltpu
from jax.experimental.pallas import tpu_sc as plsc

B = 16384
VOCAB = 1000
TAB = 1024
EMB = 128
HID = 8
NIDX = 2 * B

TILE_B = 2048
NC = 2
NS = 16
NW = NC * NS
BPW = B // NW
CH = 128
NCH = BPW // CH

GROWS = NIDX * 16 // 128
GBLK = TILE_B * 16 // 128
NBLK = B // TILE_B


def _table_body(emb_ref, w_ref, t_ref):
    e = emb_ref[...]
    w = w_ref[...]
    pa = lax.dot_general(e, w[:, :EMB], (((1,), (1,)), ((), ())),
                         preferred_element_type=jnp.float32,
                         precision=lax.Precision.HIGHEST)
    pb = lax.dot_general(e, w[:, EMB:], (((1,), (1,)), ((), ())),
                         preferred_element_type=jnp.float32,
                         precision=lax.Precision.HIGHEST)
    t_ref[...] = jnp.zeros((TAB, 16), jnp.float32)
    t_ref[:VOCAB, :] = jnp.concatenate([pa, pb], axis=1)


def _sc_gather_body(t_hbm, x0_hbm, x1_hbm, g_hbm, idx0_v, idx1_v, rows_v, sem):
    c = lax.axis_index("c")
    s = lax.axis_index("s")
    wid = s * NC + c
    pltpu.sync_copy(x0_hbm.at[pl.ds(wid * NCH, NCH)], idx0_v)
    pltpu.sync_copy(x1_hbm.at[pl.ds(wid * NCH, NCH)], idx1_v)
    copies = []
    for j in range(NCH):
        copies.append(pltpu.async_copy(
            t_hbm.at[idx0_v.at[j]], rows_v.at[pl.ds(j * CH, CH)], sem))
        copies.append(pltpu.async_copy(
            t_hbm.at[idx1_v.at[j]],
            rows_v.at[pl.ds(BPW + j * CH, CH)], sem))
    for cp in copies:
        cp.wait()
    pltpu.sync_copy(rows_v.at[pl.ds(0, BPW)], g_hbm.at[pl.ds(wid * BPW, BPW)])
    pltpu.sync_copy(rows_v.at[pl.ds(BPW, BPW)],
                    g_hbm.at[pl.ds(B + wid * BPW, BPW)])


_sc_gather = functools.partial(
    pl.kernel,
    out_type=jax.ShapeDtypeStruct((NIDX, 16), jnp.float32),
    mesh=plsc.VectorSubcoreMesh(core_axis_name="c", subcore_axis_name="s"),
    compiler_params=pltpu.CompilerParams(use_tc_tiling_on_sc=False),
    scratch_types=[
        pltpu.VMEM((NCH, CH), jnp.int32),
        pltpu.VMEM((NCH, CH), jnp.int32),
        pltpu.VMEM((2 * BPW, 16), jnp.float32),
        pltpu.SemaphoreType.DMA,
    ],
)(_sc_gather_body)


def _mlp_body(g0_ref, g1_ref, b1_ref, w_ref, b2_ref, out_ref):
    r0 = g0_ref[...]
    r1 = g1_ref[...]
    r1s = jnp.concatenate([r1[:, HID:], r1[:, :HID]], axis=1)
    q = r0 + r1s
    h8 = jnp.concatenate(
        [q[:, 16 * k:16 * k + HID] for k in range(8)], axis=0)
    hpre = h8 + b1_ref[...]
    h = jnp.tanh(hpre).astype(jnp.bfloat16)
    out_ref[...] = (
        lax.dot_general(w_ref[...], h, (((1,), (1,)), ((), ())),
                        preferred_element_type=jnp.float32)
        + b2_ref[...])


def kernel(x, emb, fc1_w, fc1_b, fc2_w, fc2_b):
    table = pl.pallas_call(
        _table_body,
        out_shape=jax.ShapeDtypeStruct((TAB, 16), jnp.float32),
    )(emb, fc1_w)

    x = x.astype(jnp.int32)
    xp = x.reshape(NBLK, 8, GBLK, 2).transpose(0, 2, 1, 3)
    x0 = xp[..., 0].reshape(B // CH, CH)
    x1 = xp[..., 1].reshape(B // CH, CH)
    g = _sc_gather(table, x0, x1).reshape(GROWS, 128)

    w2 = fc2_w.astype(jnp.bfloat16)
    b1 = fc1_b.reshape(1, HID)
    b2 = fc2_b.reshape(VOCAB, 1)
    out_t = pl.pallas_call(
        _mlp_body,
        grid=(NBLK,),
        in_specs=[
            pl.BlockSpec((GBLK, 128), lambda i: (i, 0)),
            pl.BlockSpec((GBLK, 128), lambda i: (i + NBLK, 0)),
            pl.BlockSpec((1, HID), lambda i: (0, 0)),
            pl.BlockSpec((VOCAB, HID), lambda i: (0, 0)),
            pl.BlockSpec((VOCAB, 1), lambda i: (0, 0)),
        ],
        out_specs=pl.BlockSpec((VOCAB, TILE_B), lambda i: (0, i)),
        out_shape=jax.ShapeDtypeStruct((VOCAB, B), jnp.float32),
    )(g, g, b1, w2, b2)
    return out_t.T
